# Initial kernel scaffold; baseline (speedup 1.0000x reference)
import jax
import jax.numpy as jnp
from jax.experimental import pallas as pl


def kernel(indices, embedding):
    raise NotImplementedError("write your pallas kernel here")


# speedup vs baseline: 1.6296x; 1.6296x over previous
import dataclasses

import jax
import jax.numpy as jnp
from jax import lax
from jax.experimental import pallas as pl
from jax.experimental.pallas import tpu as pltpu
from jax.experimental.pallas import tpu_sc as plsc

_W = 512
_LANES = 16


def kernel(indices, embedding):
    B, S = indices.shape
    V, H = embedding.shape
    N = B * S
    rows_per_super = 128 // H
    table128 = embedding.reshape(V // rows_per_super, 128)
    flat_idx = indices.reshape(1, N)
    hi = lax.shift_right_logical(flat_idx, 3)
    lo_col = lax.shift_left(flat_idx & 7, 4)
    mesh = plsc.VectorSubcoreMesh(core_axis_name="core", subcore_axis_name="subcore")
    cp = pltpu.CompilerParams()
    if "needs_layout_passes" in pltpu.CompilerParams.__dataclass_fields__:
        cp = dataclasses.replace(cp, needs_layout_passes=False)

    @pl.kernel(
        out_type=jax.ShapeDtypeStruct((N // 8, 128), embedding.dtype),
        mesh=mesh,
        scratch_types=[pltpu.VMEM((_W, 128), jnp.float32)],
        compiler_params=cp,
    )
    def _gather(table_hbm, hi_hbm, lo_hbm, out_hbm, buf128):
        def body(hi_vmem, lo_vmem, out_vmem):
            pltpu.sync_copy(table_hbm.at[hi_vmem.at[0]], buf128)
            iota = lax.iota(jnp.int32, _LANES)
            o_row_base = lax.shift_right_logical(iota, 3)
            o_col_base = lax.shift_left(iota & 7, 4)

            @pl.loop(0, _W // _LANES)
            def _(nb):
                n0 = nb * _LANES
                row_v = iota + n0
                lo_v = lo_vmem.at[0][pl.ds(n0, _LANES)]
                o_row = o_row_base + nb * 2
                for h in range(H):
                    val = plsc.load_gather(buf128, [row_v, lo_v + h])
                    plsc.store_scatter(out_vmem, [o_row, o_col_base + h], val)

        pltpu.emit_pipeline(
            body,
            grid=(N // _W,),
            in_specs=[
                pl.BlockSpec((1, _W), index_map=lambda i: (0, i)),
                pl.BlockSpec((1, _W), index_map=lambda i: (0, i)),
            ],
            out_specs=[pl.BlockSpec((_W // 8, 128), index_map=lambda i: (i, 0))],
            core_axis_name=("core", "subcore"),
            dimension_semantics=(pltpu.PARALLEL,),
        )(hi_hbm, lo_hbm, out_hbm)

    out = _gather(table128, hi, lo_col)
    return out.reshape(B, S, H)


# speedup vs baseline: 2.0464x; 1.2558x over previous
import dataclasses

import jax
import jax.numpy as jnp
from jax import lax
from jax.experimental import pallas as pl
from jax.experimental.pallas import tpu as pltpu
from jax.experimental.pallas import tpu_sc as plsc
from jax.experimental import layout as jex_layout

_W = 512
_LANES = 16


def kernel(indices, embedding):
    B, S = indices.shape
    V, H = embedding.shape
    N = B * S
    rows_per_super = 128 // H
    table128 = embedding.reshape(V // rows_per_super, 128)
    flat_idx = indices.reshape(1, N)
    hi = lax.shift_right_logical(flat_idx, 3)
    lo_col = lax.shift_left(flat_idx & 7, 4)
    mesh = plsc.VectorSubcoreMesh(core_axis_name="core", subcore_axis_name="subcore")
    cp = pltpu.CompilerParams()
    if "needs_layout_passes" in pltpu.CompilerParams.__dataclass_fields__:
        cp = dataclasses.replace(cp, needs_layout_passes=False)

    @pl.kernel(
        out_type=jax.ShapeDtypeStruct((N // 8, 128), embedding.dtype),
        mesh=mesh,
        scratch_types=[pltpu.VMEM((_W, 128), jnp.float32)],
        compiler_params=cp,
    )
    def _gather(table_hbm, hi_hbm, lo_hbm, out_hbm, buf128):
        def body(hi_vmem, lo_vmem, out_vmem):
            pltpu.sync_copy(table_hbm.at[hi_vmem.at[0]], buf128)
            iota = lax.iota(jnp.int32, _LANES)
            o_row_base = lax.shift_right_logical(iota, 3)
            o_col_base = lax.shift_left(iota & 7, 4)

            @pl.loop(0, _W // _LANES)
            def _(nb):
                n0 = nb * _LANES
                row_v = iota + n0
                lo_v = lo_vmem.at[0][pl.ds(n0, _LANES)]
                o_row = o_row_base + nb * 2
                for h in range(H):
                    val = plsc.load_gather(buf128, [row_v, lo_v + h])
                    plsc.store_scatter(out_vmem, [o_row, o_col_base + h], val)

        pltpu.emit_pipeline(
            body,
            grid=(N // _W,),
            in_specs=[
                pl.BlockSpec((1, _W), index_map=lambda i: (0, i)),
                pl.BlockSpec((1, _W), index_map=lambda i: (0, i)),
            ],
            out_specs=[pl.BlockSpec((_W // 8, 128), index_map=lambda i: (i, 0))],
            core_axis_name=("core", "subcore"),
            dimension_semantics=(pltpu.PARALLEL,),
        )(hi_hbm, lo_hbm, out_hbm)

    out = _gather(table128, hi, lo_col)
    out3 = out.reshape(B, S, H)
    return jex_layout.with_layout_constraint(
        out3, jex_layout.Layout(major_to_minor=(0, 1, 2))
    )
